# initial kernel scaffold (unmeasured)
import jax
import jax.numpy as jnp
from jax import lax
from jax.experimental import pallas as pl
from jax.experimental.pallas import tpu as pltpu

N_DEV = 8


def kernel(x, w_mat, scale_x, scale_w):
    m_per, k = x.shape
    n = w_mat.shape[1]
    n_per = n // N_DEV
    m_tot = m_per * N_DEV

    def body(x_ref, w_ref, sx_ref, sw_ref, out_ref,
             xb_ref, send_ref, comm_ref, send_sems, recv_sems):
        my = lax.axis_index("i")

        barrier = pltpu.get_barrier_semaphore()
        for p in range(N_DEV):
            @pl.when(my != p)
            def _():
                pl.semaphore_signal(
                    barrier, inc=1, device_id=(p,),
                    device_id_type=pl.DeviceIdType.MESH,
                )
        pl.semaphore_wait(barrier, N_DEV - 1)

        s = sx_ref[0] * sw_ref[0]
        xb_ref[...] = x_ref[...].astype(jnp.bfloat16)

        for j in range(N_DEV):
            wb = w_ref[:, j * n_per:(j + 1) * n_per].astype(jnp.bfloat16)
            acc = jnp.dot(xb_ref[...], wb, preferred_element_type=jnp.float32)
            y = acc * s
            y = y / (1.0 + jnp.exp(-jnp.clip(y, -60.0, 60.0)))
            send_ref[j] = y.astype(jnp.bfloat16)

        for j in range(N_DEV):
            @pl.when(my == j)
            def _():
                out_ref[j * m_per:(j + 1) * m_per, :] = (
                    send_ref[j].astype(jnp.float32))

        rdmas = []
        for st in range(N_DEV - 1):
            j = lax.rem(my + 1 + st, N_DEV)
            rdma = pltpu.make_async_remote_copy(
                src_ref=send_ref.at[j],
                dst_ref=comm_ref.at[st],
                send_sem=send_sems.at[st],
                recv_sem=recv_sems.at[st],
                device_id=(j,),
                device_id_type=pl.DeviceIdType.MESH,
            )
            rdma.start()
            rdmas.append(rdma)

        for st in range(N_DEV - 1):
            rdmas[st].wait_recv()
            src = lax.rem(my - 1 - st + N_DEV, N_DEV)
            out_ref[pl.ds(src * m_per, m_per), :] = (
                comm_ref[st].astype(jnp.float32))

        for st in range(N_DEV - 1):
            rdmas[st].wait_send()

    return pl.pallas_call(
        body,
        out_shape=jax.ShapeDtypeStruct((m_tot, n_per), jnp.float32),
        in_specs=[
            pl.BlockSpec(memory_space=pltpu.VMEM),
            pl.BlockSpec(memory_space=pltpu.VMEM),
            pl.BlockSpec(memory_space=pltpu.SMEM),
            pl.BlockSpec(memory_space=pltpu.SMEM),
        ],
        out_specs=pl.BlockSpec(memory_space=pltpu.VMEM),
        scratch_shapes=[
            pltpu.VMEM((m_per, k), jnp.bfloat16),
            pltpu.VMEM((N_DEV, m_per, n_per), jnp.bfloat16),
            pltpu.VMEM((N_DEV - 1, m_per, n_per), jnp.bfloat16),
            pltpu.SemaphoreType.DMA((N_DEV - 1,)),
            pltpu.SemaphoreType.DMA((N_DEV - 1,)),
        ],
        compiler_params=pltpu.CompilerParams(collective_id=0),
    )(x, w_mat, scale_x, scale_w)


# baseline (device time: 49957 ns/iter reference)
import jax
import jax.numpy as jnp
from jax import lax
from jax.experimental import pallas as pl
from jax.experimental.pallas import tpu as pltpu

N_DEV = 8


def kernel(x, w_mat, scale_x, scale_w):
    m_per, k = x.shape
    n = w_mat.shape[1]
    n_per = n // N_DEV
    m_tot = m_per * N_DEV

    def body(x_ref, w_ref, sx_ref, sw_ref, out_ref,
             xb_ref, send_ref, comm_ref, send_sems, recv_sems):
        my = lax.axis_index("i")

        barrier = pltpu.get_barrier_semaphore()
        for p in range(N_DEV):
            @pl.when(my != p)
            def _():
                pl.semaphore_signal(
                    barrier, inc=1, device_id=(p,),
                    device_id_type=pl.DeviceIdType.MESH,
                )
        pl.semaphore_wait(barrier, N_DEV - 1)

        s = sx_ref[0] * sw_ref[0]
        xb_ref[...] = x_ref[...].astype(jnp.bfloat16)

        for j in range(N_DEV):
            wb = w_ref[:, j * n_per:(j + 1) * n_per].astype(jnp.bfloat16)
            acc = jnp.dot(xb_ref[...], wb, preferred_element_type=jnp.float32)
            y = acc * s
            y = y / (1.0 + jnp.exp(-jnp.clip(y, -60.0, 60.0)))
            send_ref[j] = y.astype(jnp.bfloat16)

        for j in range(N_DEV):
            @pl.when(my == j)
            def _():
                out_ref[j * m_per:(j + 1) * m_per, :] = (
                    send_ref[j].astype(jnp.float32))

        rdmas = []
        for st in range(N_DEV - 1):
            j = lax.rem(my + 1 + st, N_DEV)
            rdma = pltpu.make_async_remote_copy(
                src_ref=send_ref.at[j],
                dst_ref=comm_ref.at[st],
                send_sem=send_sems.at[st],
                recv_sem=recv_sems.at[st],
                device_id=(j,),
                device_id_type=pl.DeviceIdType.MESH,
            )
            rdma.start()
            rdmas.append(rdma)

        for st in range(N_DEV - 1):
            rdmas[st].wait_recv()
            src = lax.rem(my - 1 - st + N_DEV, N_DEV)
            out_ref[pl.ds(src * m_per, m_per), :] = (
                comm_ref[st].astype(jnp.float32))

        for st in range(N_DEV - 1):
            rdmas[st].wait_send()

    return pl.pallas_call(
        body,
        out_shape=jax.ShapeDtypeStruct((m_tot, n_per), jnp.float32),
        in_specs=[
            pl.BlockSpec(memory_space=pltpu.VMEM),
            pl.BlockSpec(memory_space=pltpu.VMEM),
            pl.BlockSpec(memory_space=pltpu.SMEM),
            pl.BlockSpec(memory_space=pltpu.SMEM),
        ],
        out_specs=pl.BlockSpec(memory_space=pltpu.VMEM),
        scratch_shapes=[
            pltpu.VMEM((m_per, k), jnp.bfloat16),
            pltpu.VMEM((N_DEV, m_per, n_per), jnp.bfloat16),
            pltpu.VMEM((N_DEV - 1, m_per, n_per), jnp.bfloat16),
            pltpu.SemaphoreType.DMA((N_DEV - 1,)),
            pltpu.SemaphoreType.DMA((N_DEV - 1,)),
        ],
        compiler_params=pltpu.CompilerParams(
            collective_id=0,
            vmem_limit_bytes=100 * 1024 * 1024,
        ),
    )(x, w_mat, scale_x, scale_w)
